# baseline (speedup 1.0000x reference)
import functools

import jax
import jax.numpy as jnp
from jax import lax
from jax.experimental import pallas as pl
from jax.experimental.pallas import tpu as pltpu
from jax.experimental.pallas import tpu_sc as plsc

EMB_DIM = 64
NUM_CORES = 2
NUM_SUBCORES = 16
NUM_WORKERS = NUM_CORES * NUM_SUBCORES
CHUNK = 800
NBUF = 2


@functools.partial(jax.jit, static_argnames=("total",))
def _emb_gather(ids_flat, table, total):
    rows_per_worker = total // NUM_WORKERS
    n_iter = rows_per_worker // CHUNK
    n_pairs = n_iter // NBUF
    mesh = plsc.VectorSubcoreMesh(
        core_axis_name="c", subcore_axis_name="s",
        num_cores=NUM_CORES, num_subcores=NUM_SUBCORES)

    @functools.partial(
        pl.kernel,
        out_type=jax.ShapeDtypeStruct((total, EMB_DIM), jnp.float32),
        mesh=mesh,
        scratch_types=[
            pltpu.VMEM((rows_per_worker,), jnp.int32),
            [pltpu.VMEM((CHUNK, EMB_DIM), jnp.float32) for _ in range(NBUF)],
            [pltpu.SemaphoreType.DMA for _ in range(NBUF)],
            [pltpu.SemaphoreType.DMA for _ in range(NBUF)],
        ],
        compiler_params=pltpu.CompilerParams(use_tc_tiling_on_sc=False),
    )
    def gather_kernel(ids_hbm, table_hbm, out_hbm, idx_v, rows, gsem, wsem):
        wid = lax.axis_index("s") * NUM_CORES + lax.axis_index("c")
        base = wid * rows_per_worker

        pltpu.sync_copy(ids_hbm.at[pl.ds(base, rows_per_worker)], idx_v)

        def gather_cp(g, b):
            return pltpu.make_async_copy(
                table_hbm.at[idx_v.at[pl.ds(g * CHUNK, CHUNK)]],
                rows[b], gsem[b])

        def write_cp(g, b):
            return pltpu.make_async_copy(
                rows[b], out_hbm.at[pl.ds(base + g * CHUNK, CHUNK)], wsem[b])

        for b in range(NBUF):
            gather_cp(b, b).start()

        def body(p, carry):
            g0 = p * NBUF
            for b in range(NBUF):
                gather_cp(g0 + b, b).wait()
                write_cp(g0 + b, b).start()
            for b in range(NBUF):
                write_cp(g0 + b, b).wait()
                gather_cp(g0 + b + NBUF, b).start()
            return carry

        lax.fori_loop(0, n_pairs - 1, body, 0)

        g0 = (n_pairs - 1) * NBUF
        for b in range(NBUF):
            gather_cp(g0 + b, b).wait()
            write_cp(g0 + b, b).start()
        for b in range(NBUF):
            write_cp(g0 + b, b).wait()

    return gather_kernel(ids_flat, table)


def kernel(input_ids, emb_matrix):
    batch, seq = input_ids.shape
    ids_flat = input_ids.reshape(-1).astype(jnp.int32)
    out = _emb_gather(ids_flat, emb_matrix, batch * seq)
    return out.reshape(batch, seq, EMB_DIM)
